# Initial kernel scaffold; baseline (speedup 1.0000x reference)
import functools

import jax
import jax.numpy as jnp
from jax import lax
from jax.experimental import pallas as pl
from jax.experimental.pallas import tpu as pltpu
from jax.experimental.pallas import tpu_sc as plsc

N = 10000
E = 320000
D = 128
NC = 2
NS = 16
NW = NC * NS
CHUNK = 128
CPT = 79
EPT = CHUNK * CPT
E_PAD = EPT * NW
N_PAD = 10016
RPT = N_PAD // NS
DS = 16
ROWS_B = 1000

_mesh = plsc.VectorSubcoreMesh(core_axis_name="c", subcore_axis_name="s")


def _hist_body(dst3, ones_hbm, zeros_hbm, out, didx, ones_v, acc_s):
    c = lax.axis_index("c")
    s = lax.axis_index("s")
    wid = s * NC + c
    r0 = s * RPT
    pltpu.sync_copy(zeros_hbm.at[pl.ds(r0, RPT)], acc_s.at[pl.ds(r0, RPT)])
    pltpu.sync_copy(ones_hbm, ones_v)
    pltpu.sync_copy(dst3.at[wid], didx)
    plsc.subcore_barrier()

    @pl.loop(0, CPT)
    def _(j):
        pltpu.sync_copy(ones_v, acc_s.at[didx.at[j]], add=True)

    plsc.subcore_barrier()
    pltpu.sync_copy(acc_s.at[pl.ds(r0, RPT)], out.at[c, pl.ds(r0, RPT)])


_hist_kernel = functools.partial(
    pl.kernel,
    out_type=jax.ShapeDtypeStruct((NC, N_PAD, DS), jnp.float32),
    mesh=_mesh,
    scratch_types=[
        pltpu.VMEM((CPT, CHUNK), jnp.int32),
        pltpu.VMEM((CHUNK, DS), jnp.float32),
        pltpu.VMEM_SHARED((N_PAD, DS), jnp.float32),
    ],
)(_hist_body)


def _make_agg_body(d):
    del d

    def body(src3, dst3, tab, zeros_hbm, out, sidx, didx, rows, acc_s, sem):
        c = lax.axis_index("c")
        s = lax.axis_index("s")
        wid = s * NC + c
        r0 = s * RPT
        pltpu.sync_copy(zeros_hbm.at[pl.ds(r0, RPT)], acc_s.at[pl.ds(r0, RPT)])
        pltpu.sync_copy(src3.at[wid], sidx)
        pltpu.sync_copy(dst3.at[wid], didx)
        plsc.subcore_barrier()

        @pl.loop(0, CPT)
        def _(j):
            pltpu.async_copy(tab.at[sidx.at[j]], rows, sem).wait()
            pltpu.sync_copy(rows, acc_s.at[didx.at[j]], add=True)

        plsc.subcore_barrier()
        pltpu.sync_copy(acc_s.at[pl.ds(r0, RPT)], out.at[c, pl.ds(r0, RPT)])

    return body


def _make_agg_kernel(d):
    return functools.partial(
        pl.kernel,
        out_type=jax.ShapeDtypeStruct((NC, N_PAD, d), jnp.float32),
        mesh=_mesh,
        scratch_types=[
            pltpu.VMEM((CPT, CHUNK), jnp.int32),
            pltpu.VMEM((CPT, CHUNK), jnp.int32),
            pltpu.VMEM((CHUNK, d), jnp.float32),
            pltpu.VMEM_SHARED((N_PAD, d), jnp.float32),
            pltpu.SemaphoreType.DMA,
        ],
    )(_make_agg_body(d))


_agg_wide = _make_agg_kernel(D)
_agg_narrow = _make_agg_kernel(DS)


def _tc1_body(p_ref, x_ref, w_ref, y_ref, dinv_ref):
    deg = p_ref[0, :, 0:1] + p_ref[1, :, 0:1] + 1.0
    dv = lax.rsqrt(deg)
    xw = jnp.dot(x_ref[...], w_ref[...], preferred_element_type=jnp.float32)
    y_ref[...] = xw * dv
    dinv_ref[...] = jnp.broadcast_to(dv, (ROWS_B, DS))


def _tc1(p, x, W1):
    return pl.pallas_call(
        _tc1_body,
        grid=(N // ROWS_B,),
        in_specs=[
            pl.BlockSpec((NC, ROWS_B, DS), lambda i: (0, i, 0)),
            pl.BlockSpec((ROWS_B, D), lambda i: (i, 0)),
            pl.BlockSpec((D, D), lambda i: (0, 0)),
        ],
        out_specs=[
            pl.BlockSpec((ROWS_B, D), lambda i: (i, 0)),
            pl.BlockSpec((ROWS_B, DS), lambda i: (i, 0)),
        ],
        out_shape=[
            jax.ShapeDtypeStruct((N, D), jnp.float32),
            jax.ShapeDtypeStruct((N, DS), jnp.float32),
        ],
    )(p, x, W1)


def _tc2_body(p_ref, y_ref, dinv_ref, b1_ref, w2_ref, z16_ref):
    agg = p_ref[0] + p_ref[1] + y_ref[...]
    h = jnp.maximum(agg * dinv_ref[:, 0:1] + b1_ref[...], 0.0)
    z = jnp.dot(h, w2_ref[...], preferred_element_type=jnp.float32)
    z16_ref[...] = z * dinv_ref[...]


def _tc2(p, y, dinv, b1r, W2w):
    return pl.pallas_call(
        _tc2_body,
        grid=(N // ROWS_B,),
        in_specs=[
            pl.BlockSpec((NC, ROWS_B, D), lambda i: (0, i, 0)),
            pl.BlockSpec((ROWS_B, D), lambda i: (i, 0)),
            pl.BlockSpec((ROWS_B, DS), lambda i: (i, 0)),
            pl.BlockSpec((1, D), lambda i: (0, 0)),
            pl.BlockSpec((D, DS), lambda i: (0, 0)),
        ],
        out_specs=pl.BlockSpec((ROWS_B, DS), lambda i: (i, 0)),
        out_shape=jax.ShapeDtypeStruct((N, DS), jnp.float32),
    )(p, y, dinv, b1r, W2w)


def _tc3_body(p_ref, z16_ref, dinv_ref, b2_ref, out_ref):
    t = p_ref[0, :, 0:1] + p_ref[1, :, 0:1] + z16_ref[:, 0:1]
    out_ref[...] = jax.nn.sigmoid(t * dinv_ref[:, 0:1] + b2_ref[...])


def _tc3(p, z16, dinv, b2r):
    return pl.pallas_call(
        _tc3_body,
        grid=(N // ROWS_B,),
        in_specs=[
            pl.BlockSpec((NC, ROWS_B, DS), lambda i: (0, i, 0)),
            pl.BlockSpec((ROWS_B, DS), lambda i: (i, 0)),
            pl.BlockSpec((ROWS_B, DS), lambda i: (i, 0)),
            pl.BlockSpec((1, 1), lambda i: (0, 0)),
        ],
        out_specs=pl.BlockSpec((ROWS_B, 1), lambda i: (i, 0)),
        out_shape=jax.ShapeDtypeStruct((N, 1), jnp.float32),
    )(p, z16, dinv, b2r)


def kernel(x, edge_index, W1, b1, W2, b2):
    src = edge_index[0]
    dst = edge_index[1]
    pad = jnp.full((E_PAD - E,), N, dtype=jnp.int32)
    src3 = jnp.concatenate([src, pad]).reshape(NW, CPT, CHUNK)
    dst3 = jnp.concatenate([dst, pad]).reshape(NW, CPT, CHUNK)

    ones16 = jnp.ones((CHUNK, DS), jnp.float32)
    zeros16 = jnp.zeros((N_PAD, DS), jnp.float32)
    zeros128 = jnp.zeros((N_PAD, D), jnp.float32)
    row_pad = jnp.zeros((N_PAD - N, D), jnp.float32)
    row_pad16 = jnp.zeros((N_PAD - N, DS), jnp.float32)

    degp = _hist_kernel(dst3, ones16, zeros16)
    degp = degp[:, :N, :]

    y, dinv = _tc1(degp, x, W1)
    y_pad = jnp.concatenate([y, row_pad])

    aggp = _agg_wide(src3, dst3, y_pad, zeros128)
    h_in = aggp[:, :N, :]

    b1r = b1.reshape(1, D)
    W2w = jnp.concatenate([W2, jnp.zeros((D, DS - 1), jnp.float32)], axis=1)
    z16 = _tc2(h_in, y, dinv, b1r, W2w)
    z16_pad = jnp.concatenate([z16, row_pad16])

    agg2p = _agg_narrow(src3, dst3, z16_pad, zeros16)

    b2r = b2.reshape(1, 1)
    return _tc3(agg2p[:, :N, :], z16, dinv, b2r)


# speedup vs baseline: 23.4105x; 23.4105x over previous
import functools

import jax
import jax.numpy as jnp
from jax import lax
from jax.experimental import pallas as pl
from jax.experimental.pallas import tpu as pltpu
from jax.experimental.pallas import tpu_sc as plsc

N = 10000
E = 320000
D = 128
NC = 2
NS = 16
NW = NC * NS
CHUNK = 128
CPT = 79
EPT = CHUNK * CPT
E_PAD = EPT * NW
N_PAD = 10240
RPT = N_PAD // NS
ZR = 64
DS = 16
ROWS_B = 1000

_mesh = plsc.VectorSubcoreMesh(core_axis_name="c", subcore_axis_name="s")


def _zero_1d(zbuf, acc_s, r0):
    for g in range(RPT // 16):
        zbuf[pl.ds(16 * g, 16)] = jnp.zeros((16,), jnp.float32)
    pltpu.sync_copy(zbuf, acc_s.at[pl.ds(r0, RPT)])


def _out_1d(acc_s, obuf, out, c, r0):
    pltpu.sync_copy(acc_s.at[pl.ds(r0, RPT)], obuf)
    pltpu.sync_copy(obuf, out.at[pl.ds(c * N_PAD + r0, RPT)])


def _hist_body(dst3, out, didx, vbuf, obuf, acc_s):
    c = lax.axis_index("c")
    s = lax.axis_index("s")
    wid = s * NC + c
    r0 = s * RPT
    _zero_1d(obuf, acc_s, r0)
    pltpu.sync_copy(dst3.at[wid], didx)
    for g in range(CHUNK // 16):
        vbuf[pl.ds(16 * g, 16)] = jnp.ones((16,), jnp.float32)
    plsc.subcore_barrier()

    @pl.loop(0, CPT)
    def _(j):
        pltpu.sync_copy(vbuf, acc_s.at[didx.at[j]], add=True)

    plsc.subcore_barrier()
    _out_1d(acc_s, obuf, out, c, r0)


_hist_kernel = functools.partial(
    pl.kernel,
    out_type=jax.ShapeDtypeStruct((NC * N_PAD,), jnp.float32),
    mesh=_mesh,
    compiler_params=pltpu.CompilerParams(needs_layout_passes=False),
    scratch_types=[
        pltpu.VMEM((CPT, CHUNK), jnp.int32),
        pltpu.VMEM((CHUNK,), jnp.float32),
        pltpu.VMEM((RPT,), jnp.float32),
        pltpu.VMEM_SHARED((N_PAD,), jnp.float32),
    ],
)(_hist_body)


def _narrow_body(src3, dst3, tab, out, sidx, didx, tab_v, vbuf, obuf, acc_s):
    c = lax.axis_index("c")
    s = lax.axis_index("s")
    wid = s * NC + c
    r0 = s * RPT
    _zero_1d(obuf, acc_s, r0)
    pltpu.sync_copy(src3.at[wid], sidx)
    pltpu.sync_copy(dst3.at[wid], didx)
    pltpu.sync_copy(tab, tab_v)
    plsc.subcore_barrier()

    @pl.loop(0, CPT)
    def _(j):
        for g in range(CHUNK // 16):
            src_v = sidx[j, pl.ds(16 * g, 16)]
            vbuf[pl.ds(16 * g, 16)] = plsc.load_gather(tab_v, [src_v])
        pltpu.sync_copy(vbuf, acc_s.at[didx.at[j]], add=True)

    plsc.subcore_barrier()
    _out_1d(acc_s, obuf, out, c, r0)


_agg_narrow = functools.partial(
    pl.kernel,
    out_type=jax.ShapeDtypeStruct((NC * N_PAD,), jnp.float32),
    mesh=_mesh,
    compiler_params=pltpu.CompilerParams(needs_layout_passes=False),
    scratch_types=[
        pltpu.VMEM((CPT, CHUNK), jnp.int32),
        pltpu.VMEM((CPT, CHUNK), jnp.int32),
        pltpu.VMEM((N_PAD,), jnp.float32),
        pltpu.VMEM((CHUNK,), jnp.float32),
        pltpu.VMEM((RPT,), jnp.float32),
        pltpu.VMEM_SHARED((N_PAD,), jnp.float32),
    ],
)(_narrow_body)


def _wide_body(src3, dst3, tab, out, sidx, didx, rows, sbuf, acc_s, sem):
    c = lax.axis_index("c")
    s = lax.axis_index("s")
    wid = s * NC + c
    r0 = s * RPT

    @pl.loop(0, ZR)
    def _(i):
        for g in range(D // 16):
            sbuf[i, pl.ds(16 * g, 16)] = jnp.zeros((16,), jnp.float32)

    @pl.loop(0, RPT // ZR)
    def _(k):
        pltpu.sync_copy(sbuf, acc_s.at[pl.ds(r0 + ZR * k, ZR)])

    pltpu.sync_copy(src3.at[wid], sidx)
    pltpu.sync_copy(dst3.at[wid], didx)
    plsc.subcore_barrier()

    @pl.loop(0, CPT)
    def _(j):
        pltpu.async_copy(tab.at[sidx.at[j]], rows, sem).wait()
        pltpu.sync_copy(rows, acc_s.at[didx.at[j]], add=True)

    plsc.subcore_barrier()

    @pl.loop(0, RPT // ZR)
    def _(k):
        pltpu.sync_copy(acc_s.at[pl.ds(r0 + ZR * k, ZR)], sbuf)
        pltpu.sync_copy(sbuf, out.at[c, pl.ds(r0 + ZR * k, ZR)])


_agg_wide = functools.partial(
    pl.kernel,
    out_type=jax.ShapeDtypeStruct((NC, N_PAD, D), jnp.float32),
    mesh=_mesh,
    compiler_params=pltpu.CompilerParams(needs_layout_passes=False),
    scratch_types=[
        pltpu.VMEM((CPT, CHUNK), jnp.int32),
        pltpu.VMEM((CPT, CHUNK), jnp.int32),
        pltpu.VMEM((CHUNK, D), jnp.float32),
        pltpu.VMEM((ZR, D), jnp.float32),
        pltpu.VMEM_SHARED((N_PAD, D), jnp.float32),
        pltpu.SemaphoreType.DMA,
    ],
)(_wide_body)


def _tc1_body(p_ref, x_ref, w_ref, y_ref, dinv_ref):
    deg = p_ref[0] + p_ref[1] + 1.0
    dv = lax.rsqrt(deg)
    xw = jnp.dot(x_ref[...], w_ref[...], preferred_element_type=jnp.float32)
    y_ref[...] = xw * dv
    dinv_ref[...] = jnp.broadcast_to(dv, (ROWS_B, DS))


def _tc1(p, x, W1):
    return pl.pallas_call(
        _tc1_body,
        grid=(N // ROWS_B,),
        in_specs=[
            pl.BlockSpec((NC, ROWS_B, 1), lambda i: (0, i, 0)),
            pl.BlockSpec((ROWS_B, D), lambda i: (i, 0)),
            pl.BlockSpec((D, D), lambda i: (0, 0)),
        ],
        out_specs=[
            pl.BlockSpec((ROWS_B, D), lambda i: (i, 0)),
            pl.BlockSpec((ROWS_B, DS), lambda i: (i, 0)),
        ],
        out_shape=[
            jax.ShapeDtypeStruct((N, D), jnp.float32),
            jax.ShapeDtypeStruct((N, DS), jnp.float32),
        ],
    )(p, x, W1)


def _tc2_body(p_ref, y_ref, dinv_ref, b1_ref, w2_ref, z16_ref):
    agg = p_ref[0] + p_ref[1] + y_ref[...]
    h = jnp.maximum(agg * dinv_ref[:, 0:1] + b1_ref[...], 0.0)
    z = jnp.dot(h, w2_ref[...], preferred_element_type=jnp.float32)
    z16_ref[...] = z * dinv_ref[...]


def _tc2(p, y, dinv, b1r, W2w):
    return pl.pallas_call(
        _tc2_body,
        grid=(N // ROWS_B,),
        in_specs=[
            pl.BlockSpec((NC, ROWS_B, D), lambda i: (0, i, 0)),
            pl.BlockSpec((ROWS_B, D), lambda i: (i, 0)),
            pl.BlockSpec((ROWS_B, DS), lambda i: (i, 0)),
            pl.BlockSpec((1, D), lambda i: (0, 0)),
            pl.BlockSpec((D, DS), lambda i: (0, 0)),
        ],
        out_specs=pl.BlockSpec((ROWS_B, DS), lambda i: (i, 0)),
        out_shape=jax.ShapeDtypeStruct((N, DS), jnp.float32),
    )(p, y, dinv, b1r, W2w)


def _tc3_body(p_ref, z16_ref, dinv_ref, b2_ref, out_ref):
    t = p_ref[0] + p_ref[1] + z16_ref[:, 0:1]
    out_ref[...] = jax.nn.sigmoid(t * dinv_ref[:, 0:1] + b2_ref[...])


def _tc3(p, z16, dinv, b2r):
    return pl.pallas_call(
        _tc3_body,
        grid=(N // ROWS_B,),
        in_specs=[
            pl.BlockSpec((NC, ROWS_B, 1), lambda i: (0, i, 0)),
            pl.BlockSpec((ROWS_B, DS), lambda i: (i, 0)),
            pl.BlockSpec((ROWS_B, DS), lambda i: (i, 0)),
            pl.BlockSpec((1, 1), lambda i: (0, 0)),
        ],
        out_specs=pl.BlockSpec((ROWS_B, 1), lambda i: (i, 0)),
        out_shape=jax.ShapeDtypeStruct((N, 1), jnp.float32),
    )(p, z16, dinv, b2r)


def kernel(x, edge_index, W1, b1, W2, b2):
    src = edge_index[0]
    dst = edge_index[1]
    pad = jnp.full((E_PAD - E,), N, dtype=jnp.int32)
    src3 = jnp.concatenate([src, pad]).reshape(NW, CPT, CHUNK)
    dst3 = jnp.concatenate([dst, pad]).reshape(NW, CPT, CHUNK)

    row_pad = jnp.zeros((N_PAD - N, D), jnp.float32)

    degp = _hist_kernel(dst3)
    degp = degp.reshape(NC, N_PAD)[:, :N, None]

    y, dinv = _tc1(degp, x, W1)
    y_pad = jnp.concatenate([y, row_pad])

    aggp = _agg_wide(src3, dst3, y_pad)
    h_in = aggp[:, :N, :]

    b1r = b1.reshape(1, D)
    W2w = jnp.concatenate([W2, jnp.zeros((D, DS - 1), jnp.float32)], axis=1)
    z16 = _tc2(h_in, y, dinv, b1r, W2w)
    z_flat = jnp.concatenate([z16[:, 0], jnp.zeros((N_PAD - N,), jnp.float32)])

    agg2p = _agg_narrow(src3, dst3, z_flat)
    agg2p = agg2p.reshape(NC, N_PAD)[:, :N, None]

    b2r = b2.reshape(1, 1)
    return _tc3(agg2p, z16, dinv, b2r)


# speedup vs baseline: 43.6096x; 1.8628x over previous
import functools

import jax
import jax.numpy as jnp
from jax import lax
from jax.experimental import pallas as pl
from jax.experimental.pallas import tpu as pltpu
from jax.experimental.pallas import tpu_sc as plsc

N = 10000
E = 320000
D = 128
NC = 2
NS = 16
NW = NC * NS
CHUNK = 128
CPT = 80
HALF = CPT // 2
EPT = CHUNK * CPT
E_PAD = EPT * NW
N_PAD = 10240
RPT = N_PAD // NS
ZR = 128
DS = 16
ROWS_B = 1000

_mesh = plsc.VectorSubcoreMesh(core_axis_name="c", subcore_axis_name="s")


def _zero_1d(zbuf, acc_s, r0):
    for g in range(RPT // 16):
        zbuf[pl.ds(16 * g, 16)] = jnp.zeros((16,), jnp.float32)
    pltpu.sync_copy(zbuf, acc_s.at[pl.ds(r0, RPT)])


def _out_1d(acc_s, obuf, out, c, r0):
    pltpu.sync_copy(acc_s.at[pl.ds(r0, RPT)], obuf)
    pltpu.sync_copy(obuf, out.at[pl.ds(c * N_PAD + r0, RPT)])


def _hist_body(dst3, out, didx, vbuf, obuf, acc_s):
    c = lax.axis_index("c")
    s = lax.axis_index("s")
    wid = s * NC + c
    r0 = s * RPT
    _zero_1d(obuf, acc_s, r0)
    pltpu.sync_copy(dst3.at[wid], didx)
    for g in range(CHUNK // 16):
        vbuf[pl.ds(16 * g, 16)] = jnp.ones((16,), jnp.float32)
    plsc.subcore_barrier()

    @pl.loop(0, CPT)
    def _(j):
        pltpu.sync_copy(vbuf, acc_s.at[didx.at[j]], add=True)

    plsc.subcore_barrier()
    _out_1d(acc_s, obuf, out, c, r0)


_hist_kernel = functools.partial(
    pl.kernel,
    out_type=jax.ShapeDtypeStruct((NC * N_PAD,), jnp.float32),
    mesh=_mesh,
    compiler_params=pltpu.CompilerParams(needs_layout_passes=False),
    scratch_types=[
        pltpu.VMEM((CPT, CHUNK), jnp.int32),
        pltpu.VMEM((CHUNK,), jnp.float32),
        pltpu.VMEM((RPT,), jnp.float32),
        pltpu.VMEM_SHARED((N_PAD,), jnp.float32),
    ],
)(_hist_body)


def _narrow_body(src3, dst3, tab, out, sidx, didx, tab_v, vbuf, obuf, acc_s):
    c = lax.axis_index("c")
    s = lax.axis_index("s")
    wid = s * NC + c
    r0 = s * RPT
    _zero_1d(obuf, acc_s, r0)
    pltpu.sync_copy(src3.at[wid], sidx)
    pltpu.sync_copy(dst3.at[wid], didx)
    pltpu.sync_copy(tab, tab_v)
    plsc.subcore_barrier()

    @pl.loop(0, CPT)
    def _(j):
        for g in range(CHUNK // 16):
            src_v = sidx[j, pl.ds(16 * g, 16)]
            vbuf[pl.ds(16 * g, 16)] = plsc.load_gather(tab_v, [src_v])
        pltpu.sync_copy(vbuf, acc_s.at[didx.at[j]], add=True)

    plsc.subcore_barrier()
    _out_1d(acc_s, obuf, out, c, r0)


_agg_narrow = functools.partial(
    pl.kernel,
    out_type=jax.ShapeDtypeStruct((NC * N_PAD,), jnp.float32),
    mesh=_mesh,
    compiler_params=pltpu.CompilerParams(needs_layout_passes=False),
    scratch_types=[
        pltpu.VMEM((CPT, CHUNK), jnp.int32),
        pltpu.VMEM((CPT, CHUNK), jnp.int32),
        pltpu.VMEM((N_PAD,), jnp.float32),
        pltpu.VMEM((CHUNK,), jnp.float32),
        pltpu.VMEM((RPT,), jnp.float32),
        pltpu.VMEM_SHARED((N_PAD,), jnp.float32),
    ],
)(_narrow_body)


def _wide_body(src3, dst3, tab, out, sidx, didx, rows0, rows1, acc_s,
               sem0, sem1):
    c = lax.axis_index("c")
    s = lax.axis_index("s")
    wid = s * NC + c
    r0 = s * RPT
    sbuf = rows0.at[pl.ds(0, ZR)]

    @pl.loop(0, ZR)
    def _(i):
        for g in range(D // 16):
            rows0[i, pl.ds(16 * g, 16)] = jnp.zeros((16,), jnp.float32)

    @pl.loop(0, RPT // ZR)
    def _(k):
        pltpu.sync_copy(sbuf, acc_s.at[pl.ds(r0 + ZR * k, ZR)])

    plsc.subcore_barrier()

    for h in range(2):
        pltpu.sync_copy(src3.at[wid, pl.ds(h * HALF, HALF)], sidx)
        pltpu.sync_copy(dst3.at[wid, pl.ds(h * HALF, HALF)], didx)
        pltpu.async_copy(tab.at[sidx.at[0]], rows0, sem0)

        @pl.loop(0, HALF // 2 - 1)
        def _(k):
            j = 2 * k
            pltpu.async_copy(tab.at[sidx.at[j + 1]], rows1, sem1)
            pltpu.make_async_copy(tab.at[sidx.at[j]], rows0, sem0).wait()
            pltpu.sync_copy(rows0, acc_s.at[didx.at[j]], add=True)
            pltpu.async_copy(tab.at[sidx.at[j + 2]], rows0, sem0)
            pltpu.make_async_copy(tab.at[sidx.at[j + 1]], rows1, sem1).wait()
            pltpu.sync_copy(rows1, acc_s.at[didx.at[j + 1]], add=True)

        pltpu.async_copy(tab.at[sidx.at[HALF - 1]], rows1, sem1)
        pltpu.make_async_copy(tab.at[sidx.at[HALF - 2]], rows0, sem0).wait()
        pltpu.sync_copy(rows0, acc_s.at[didx.at[HALF - 2]], add=True)
        pltpu.make_async_copy(tab.at[sidx.at[HALF - 1]], rows1, sem1).wait()
        pltpu.sync_copy(rows1, acc_s.at[didx.at[HALF - 1]], add=True)

    plsc.subcore_barrier()

    @pl.loop(0, RPT // ZR)
    def _(k):
        pltpu.sync_copy(acc_s.at[pl.ds(r0 + ZR * k, ZR)], sbuf)
        pltpu.sync_copy(sbuf, out.at[c, pl.ds(r0 + ZR * k, ZR)])


_agg_wide = functools.partial(
    pl.kernel,
    out_type=jax.ShapeDtypeStruct((NC, N_PAD, D), jnp.float32),
    mesh=_mesh,
    compiler_params=pltpu.CompilerParams(needs_layout_passes=False),
    scratch_types=[
        pltpu.VMEM((HALF, CHUNK), jnp.int32),
        pltpu.VMEM((HALF, CHUNK), jnp.int32),
        pltpu.VMEM((CHUNK, D), jnp.float32),
        pltpu.VMEM((CHUNK, D), jnp.float32),
        pltpu.VMEM_SHARED((N_PAD, D), jnp.float32),
        pltpu.SemaphoreType.DMA,
        pltpu.SemaphoreType.DMA,
    ],
)(_wide_body)


def _tc1_body(p_ref, x_ref, w_ref, y_ref, dinv_ref):
    deg = p_ref[0] + p_ref[1] + 1.0
    dv = lax.rsqrt(deg)
    xw = jnp.dot(x_ref[...], w_ref[...], preferred_element_type=jnp.float32)
    y_ref[...] = xw * dv
    dinv_ref[...] = jnp.broadcast_to(dv, (ROWS_B, DS))


def _tc1(p, x, W1):
    return pl.pallas_call(
        _tc1_body,
        grid=(N // ROWS_B,),
        in_specs=[
            pl.BlockSpec((NC, ROWS_B, 1), lambda i: (0, i, 0)),
            pl.BlockSpec((ROWS_B, D), lambda i: (i, 0)),
            pl.BlockSpec((D, D), lambda i: (0, 0)),
        ],
        out_specs=[
            pl.BlockSpec((ROWS_B, D), lambda i: (i, 0)),
            pl.BlockSpec((ROWS_B, DS), lambda i: (i, 0)),
        ],
        out_shape=[
            jax.ShapeDtypeStruct((N, D), jnp.float32),
            jax.ShapeDtypeStruct((N, DS), jnp.float32),
        ],
    )(p, x, W1)


def _tc2_body(p_ref, y_ref, dinv_ref, b1_ref, w2_ref, z16_ref):
    agg = p_ref[0] + p_ref[1] + y_ref[...]
    h = jnp.maximum(agg * dinv_ref[:, 0:1] + b1_ref[...], 0.0)
    z = jnp.dot(h, w2_ref[...], preferred_element_type=jnp.float32)
    z16_ref[...] = z * dinv_ref[...]


def _tc2(p, y, dinv, b1r, W2w):
    return pl.pallas_call(
        _tc2_body,
        grid=(N // ROWS_B,),
        in_specs=[
            pl.BlockSpec((NC, ROWS_B, D), lambda i: (0, i, 0)),
            pl.BlockSpec((ROWS_B, D), lambda i: (i, 0)),
            pl.BlockSpec((ROWS_B, DS), lambda i: (i, 0)),
            pl.BlockSpec((1, D), lambda i: (0, 0)),
            pl.BlockSpec((D, DS), lambda i: (0, 0)),
        ],
        out_specs=pl.BlockSpec((ROWS_B, DS), lambda i: (i, 0)),
        out_shape=jax.ShapeDtypeStruct((N, DS), jnp.float32),
    )(p, y, dinv, b1r, W2w)


def _tc3_body(p_ref, z16_ref, dinv_ref, b2_ref, out_ref):
    t = p_ref[0] + p_ref[1] + z16_ref[:, 0:1]
    out_ref[...] = jax.nn.sigmoid(t * dinv_ref[:, 0:1] + b2_ref[...])


def _tc3(p, z16, dinv, b2r):
    return pl.pallas_call(
        _tc3_body,
        grid=(N // ROWS_B,),
        in_specs=[
            pl.BlockSpec((NC, ROWS_B, 1), lambda i: (0, i, 0)),
            pl.BlockSpec((ROWS_B, DS), lambda i: (i, 0)),
            pl.BlockSpec((ROWS_B, DS), lambda i: (i, 0)),
            pl.BlockSpec((1, 1), lambda i: (0, 0)),
        ],
        out_specs=pl.BlockSpec((ROWS_B, 1), lambda i: (i, 0)),
        out_shape=jax.ShapeDtypeStruct((N, 1), jnp.float32),
    )(p, z16, dinv, b2r)


def kernel(x, edge_index, W1, b1, W2, b2):
    src = edge_index[0]
    dst = edge_index[1]
    pad = (jnp.arange(E_PAD - E, dtype=jnp.int32) % (N_PAD - N)) + N
    src3 = jnp.concatenate([src, pad]).reshape(NW, CPT, CHUNK)
    dst3 = jnp.concatenate([dst, pad]).reshape(NW, CPT, CHUNK)

    row_pad = jnp.zeros((N_PAD - N, D), jnp.float32)

    degp = _hist_kernel(dst3)
    degp = degp.reshape(NC, N_PAD)[:, :N, None]

    y, dinv = _tc1(degp, x, W1)
    y_pad = jnp.concatenate([y, row_pad])

    aggp = _agg_wide(src3, dst3, y_pad)
    h_in = aggp[:, :N, :]

    b1r = b1.reshape(1, D)
    W2w = jnp.concatenate([W2, jnp.zeros((D, DS - 1), jnp.float32)], axis=1)
    z16 = _tc2(h_in, y, dinv, b1r, W2w)
    z_flat = jnp.concatenate([z16[:, 0], jnp.zeros((N_PAD - N,), jnp.float32)])

    agg2p = _agg_narrow(src3, dst3, z_flat)
    agg2p = agg2p.reshape(NC, N_PAD)[:, :N, None]

    b2r = b2.reshape(1, 1)
    return _tc3(agg2p, z16, dinv, b2r)
